# baseline (speedup 1.0000x reference)
import functools

import jax
import jax.numpy as jnp
from jax.experimental import pallas as pl
from jax.experimental.pallas import tpu as pltpu


def _router_kernel(x_ref, w1_ref, b1_ref, w2_ref, b2_ref, out_ref,
                   h0, h1, *, k_steps, rows):
    m = pl.program_id(0)
    k = pl.program_id(1)

    def _body(h_acc, h_prev):
        @pl.when(k == 0)
        def _init_h():
            h_acc[...] = jnp.zeros_like(h_acc)

        h_acc[...] += jnp.dot(x_ref[...], w1_ref[...],
                              preferred_element_type=jnp.float32)

        h = h_prev[pl.ds(k * rows, rows), :] + b1_ref[...]
        hg = 0.5 * h * (1.0 + jax.lax.erf(h * 0.7071067811865476))
        logits = jnp.dot(hg, w2_ref[...],
                         preferred_element_type=jnp.float32) + b2_ref[...]
        mx = jnp.max(logits, axis=1, keepdims=True)
        e = jnp.exp(logits - mx)
        out_ref[...] = e / jnp.sum(e, axis=1, keepdims=True)

    @pl.when(jax.lax.rem(m, 2) == 0)
    def _even():
        _body(h0, h1)

    @pl.when(jax.lax.rem(m, 2) == 1)
    def _odd():
        _body(h1, h0)


@jax.jit
def kernel(x, W1, b1, W2, b2):
    M, K = x.shape
    _, N = W1.shape
    E = W2.shape[1]

    bm = min(1024, M)
    bk = min(512, K)
    m_steps = M // bm
    k_steps = K // bk
    rows = bm // k_steps
    grid = (m_steps + 1, k_steps)
    last_m = m_steps - 1

    b1r = b1.reshape(1, N)
    b2r = b2.reshape(1, E)

    out_padded = pl.pallas_call(
        functools.partial(_router_kernel, k_steps=k_steps, rows=rows),
        grid=grid,
        in_specs=[
            pl.BlockSpec((bm, bk), lambda m, k: (jnp.minimum(m, last_m), k)),
            pl.BlockSpec((bk, N), lambda m, k: (k, 0)),
            pl.BlockSpec((1, N), lambda m, k: (0, 0)),
            pl.BlockSpec((N, E), lambda m, k: (0, 0)),
            pl.BlockSpec((1, E), lambda m, k: (0, 0)),
        ],
        out_specs=pl.BlockSpec(
            (rows, E),
            lambda m, k: (m * k_steps + k, 0)),
        out_shape=jax.ShapeDtypeStruct((M + bm, E), jnp.float32),
        scratch_shapes=[
            pltpu.VMEM((bm, N), jnp.float32),
            pltpu.VMEM((bm, N), jnp.float32),
        ],
        compiler_params=pltpu.CompilerParams(
            dimension_semantics=("arbitrary", "arbitrary")),
    )(x, W1, b1r, W2, b2r)
    return out_padded[bm:, :]


# speedup vs baseline: 1.0967x; 1.0967x over previous
import functools

import jax
import jax.numpy as jnp
from jax.experimental import pallas as pl
from jax.experimental.pallas import tpu as pltpu


def _router_kernel(x_ref, w1_ref, b1_ref, w2_ref, b2_ref, out_ref,
                   h_acc, logits_acc, *, n_steps, k_steps):
    n = pl.program_id(1)
    k = pl.program_id(2)

    @pl.when(k == 0)
    def _init_h():
        h_acc[...] = jnp.zeros_like(h_acc)

    h_acc[...] += jnp.dot(x_ref[...], w1_ref[...],
                          preferred_element_type=jnp.float32)

    @pl.when(k == k_steps - 1)
    def _epilogue():
        h = h_acc[...] + b1_ref[...]
        hg = 0.5 * h * (1.0 + jax.lax.erf(h * 0.7071067811865476))
        part = jnp.dot(hg, w2_ref[...], preferred_element_type=jnp.float32)

        @pl.when(n == 0)
        def _init_logits():
            logits_acc[...] = part + b2_ref[...]

        @pl.when(n > 0)
        def _acc_logits():
            logits_acc[...] += part

        @pl.when(n == n_steps - 1)
        def _softmax():
            logits = logits_acc[...]
            mx = jnp.max(logits, axis=1, keepdims=True)
            e = jnp.exp(logits - mx)
            out_ref[...] = e / jnp.sum(e, axis=1, keepdims=True)


@jax.jit
def kernel(x, W1, b1, W2, b2):
    M, K = x.shape
    _, N = W1.shape
    E = W2.shape[1]

    bm = min(1024, M)
    bn = min(4096, N)
    bk = min(512, K)
    grid = (M // bm, N // bn, K // bk)

    b1r = b1.reshape(1, N)
    b2r = b2.reshape(1, E)

    return pl.pallas_call(
        functools.partial(_router_kernel, n_steps=grid[1], k_steps=grid[2]),
        grid=grid,
        in_specs=[
            pl.BlockSpec((bm, bk), lambda m, n, k: (m, k)),
            pl.BlockSpec((bk, bn), lambda m, n, k: (k, n)),
            pl.BlockSpec((1, bn), lambda m, n, k: (0, n)),
            pl.BlockSpec((bn, E), lambda m, n, k: (n, 0)),
            pl.BlockSpec((1, E), lambda m, n, k: (0, 0)),
        ],
        out_specs=pl.BlockSpec((bm, E), lambda m, n, k: (m, 0)),
        out_shape=jax.ShapeDtypeStruct((M, E), jnp.float32),
        scratch_shapes=[
            pltpu.VMEM((bm, bn), jnp.float32),
            pltpu.VMEM((bm, E), jnp.float32),
        ],
        compiler_params=pltpu.CompilerParams(
            dimension_semantics=("parallel", "arbitrary", "arbitrary")),
    )(x, W1, b1r, W2, b2r)


# speedup vs baseline: 1.0976x; 1.0009x over previous
import functools

import jax
import jax.numpy as jnp
from jax.experimental import pallas as pl
from jax.experimental.pallas import tpu as pltpu


def _router_kernel(x_ref, w1_ref, b1_ref, w2_ref, b2_ref, out_ref,
                   h_acc, logits_acc, *, n_steps, k_steps):
    n = pl.program_id(1)
    k = pl.program_id(2)

    @pl.when(k == 0)
    def _init_h():
        h_acc[...] = jnp.zeros_like(h_acc)

    h_acc[...] += jnp.dot(x_ref[...], w1_ref[...],
                          preferred_element_type=jnp.float32)

    @pl.when(k == k_steps - 1)
    def _epilogue():
        h = h_acc[...] + b1_ref[...]
        hg = 0.5 * h * (1.0 + jax.lax.erf(h * 0.7071067811865476))
        part = jnp.dot(hg, w2_ref[...], preferred_element_type=jnp.float32)

        if n_steps == 1:
            logits = part + b2_ref[...]
            mx = jnp.max(logits, axis=1, keepdims=True)
            e = jnp.exp(logits - mx)
            out_ref[...] = e / jnp.sum(e, axis=1, keepdims=True)
            return

        @pl.when(n == 0)
        def _init_logits():
            logits_acc[...] = part + b2_ref[...]

        @pl.when(n > 0)
        def _acc_logits():
            logits_acc[...] += part

        @pl.when(n == n_steps - 1)
        def _softmax():
            logits = logits_acc[...]
            mx = jnp.max(logits, axis=1, keepdims=True)
            e = jnp.exp(logits - mx)
            out_ref[...] = e / jnp.sum(e, axis=1, keepdims=True)


@jax.jit
def kernel(x, W1, b1, W2, b2):
    M, K = x.shape
    _, N = W1.shape
    E = W2.shape[1]

    bm = min(1024, M)
    bn = min(4096, N)
    bk = min(512, K)
    grid = (M // bm, N // bn, K // bk)

    b1r = b1.reshape(1, N)
    b2r = b2.reshape(1, E)

    return pl.pallas_call(
        functools.partial(_router_kernel, n_steps=grid[1], k_steps=grid[2]),
        grid=grid,
        in_specs=[
            pl.BlockSpec((bm, bk), lambda m, n, k: (m, k)),
            pl.BlockSpec((bk, bn), lambda m, n, k: (k, n)),
            pl.BlockSpec((1, bn), lambda m, n, k: (0, n)),
            pl.BlockSpec((bn, E), lambda m, n, k: (n, 0)),
            pl.BlockSpec((1, E), lambda m, n, k: (0, 0)),
        ],
        out_specs=pl.BlockSpec((bm, E), lambda m, n, k: (m, 0)),
        out_shape=jax.ShapeDtypeStruct((M, E), jnp.float32),
        scratch_shapes=[
            pltpu.VMEM((bm, bn), jnp.float32),
            pltpu.VMEM((bm, E), jnp.float32),
        ],
        compiler_params=pltpu.CompilerParams(
            dimension_semantics=("parallel", "arbitrary", "arbitrary")),
    )(x, W1, b1r, W2, b2r)


# speedup vs baseline: 1.1243x; 1.0243x over previous
import functools

import jax
import jax.numpy as jnp
from jax.experimental import pallas as pl
from jax.experimental.pallas import tpu as pltpu


def _router_kernel(x_ref, w1_ref, b1_ref, w2_ref, b2_ref, out_ref,
                   h_acc, logits_acc, *, n_steps, k_steps):
    n = pl.program_id(1)
    k = pl.program_id(2)

    @pl.when(k == 0)
    def _init_h():
        h_acc[...] = jnp.zeros_like(h_acc)

    @pl.when(k < k_steps - 1)
    def _matmul():
        h_acc[...] += jnp.dot(x_ref[...], w1_ref[...],
                              preferred_element_type=jnp.float32)

    @pl.when(k == k_steps - 1)
    def _epilogue():
        h = h_acc[...] + jnp.dot(x_ref[...], w1_ref[...],
                                 preferred_element_type=jnp.float32) + b1_ref[...]
        hg = 0.5 * h * (1.0 + jax.lax.erf(h * 0.7071067811865476))
        part = jnp.dot(hg, w2_ref[...], preferred_element_type=jnp.float32)

        if n_steps == 1:
            logits = part + b2_ref[...]
            mx = jnp.max(logits, axis=1, keepdims=True)
            e = jnp.exp(logits - mx)
            out_ref[...] = e / jnp.sum(e, axis=1, keepdims=True)
            return

        @pl.when(n == 0)
        def _init_logits():
            logits_acc[...] = part + b2_ref[...]

        @pl.when(n > 0)
        def _acc_logits():
            logits_acc[...] += part

        @pl.when(n == n_steps - 1)
        def _softmax():
            logits = logits_acc[...]
            mx = jnp.max(logits, axis=1, keepdims=True)
            e = jnp.exp(logits - mx)
            out_ref[...] = e / jnp.sum(e, axis=1, keepdims=True)


@jax.jit
def kernel(x, W1, b1, W2, b2):
    M, K = x.shape
    _, N = W1.shape
    E = W2.shape[1]

    bm = min(1024, M)
    bn = min(4096, N)
    bk = min(512, K)
    grid = (M // bm, N // bn, K // bk)

    b1r = b1.reshape(1, N)
    b2r = b2.reshape(1, E)

    return pl.pallas_call(
        functools.partial(_router_kernel, n_steps=grid[1], k_steps=grid[2]),
        grid=grid,
        in_specs=[
            pl.BlockSpec((bm, bk), lambda m, n, k: (m, k)),
            pl.BlockSpec((bk, bn), lambda m, n, k: (k, n)),
            pl.BlockSpec((1, bn), lambda m, n, k: (0, n)),
            pl.BlockSpec((bn, E), lambda m, n, k: (n, 0)),
            pl.BlockSpec((1, E), lambda m, n, k: (0, 0)),
        ],
        out_specs=pl.BlockSpec((bm, E), lambda m, n, k: (m, 0)),
        out_shape=jax.ShapeDtypeStruct((M, E), jnp.float32),
        scratch_shapes=[
            pltpu.VMEM((bm, bn), jnp.float32),
            pltpu.VMEM((bm, E), jnp.float32),
        ],
        compiler_params=pltpu.CompilerParams(
            dimension_semantics=("parallel", "arbitrary", "arbitrary")),
    )(x, W1, b1r, W2, b2r)


# speedup vs baseline: 1.1533x; 1.0258x over previous
import functools

import jax
import jax.numpy as jnp
from jax.experimental import pallas as pl
from jax.experimental.pallas import tpu as pltpu


def _router_kernel(x_ref, w1_ref, b1_ref, w2_ref, b2_ref, out_ref,
                   h_acc, logits_acc, *, n_steps, k_steps):
    n = pl.program_id(1)
    k = pl.program_id(2)

    if k_steps > 1:
        @pl.when(k == 0)
        def _init_h():
            h_acc[...] = jnp.dot(x_ref[...], w1_ref[...],
                                 preferred_element_type=jnp.float32)

        @pl.when((k > 0) & (k < k_steps - 1))
        def _matmul():
            h_acc[...] += jnp.dot(x_ref[...], w1_ref[...],
                                  preferred_element_type=jnp.float32)

    @pl.when(k == k_steps - 1)
    def _epilogue():
        hd = jnp.dot(x_ref[...], w1_ref[...],
                     preferred_element_type=jnp.float32)
        if k_steps > 1:
            hd = h_acc[...] + hd
        h = hd + b1_ref[...]
        hg = 0.5 * h * (1.0 + jax.lax.erf(h * 0.7071067811865476))
        part = jnp.dot(hg, w2_ref[...], preferred_element_type=jnp.float32)

        if n_steps == 1:
            logits = part + b2_ref[...]
            mx = jnp.max(logits, axis=1, keepdims=True)
            e = jnp.exp(logits - mx)
            out_ref[...] = e / jnp.sum(e, axis=1, keepdims=True)
            return

        @pl.when(n == 0)
        def _init_logits():
            logits_acc[...] = part + b2_ref[...]

        @pl.when(n > 0)
        def _acc_logits():
            logits_acc[...] += part

        @pl.when(n == n_steps - 1)
        def _softmax():
            logits = logits_acc[...]
            mx = jnp.max(logits, axis=1, keepdims=True)
            e = jnp.exp(logits - mx)
            out_ref[...] = e / jnp.sum(e, axis=1, keepdims=True)


@jax.jit
def kernel(x, W1, b1, W2, b2):
    M, K = x.shape
    _, N = W1.shape
    E = W2.shape[1]

    bm = min(1024, M)
    bn = min(4096, N)
    bk = min(512, K)
    grid = (M // bm, N // bn, K // bk)

    b1r = b1.reshape(1, N)
    b2r = b2.reshape(1, E)

    return pl.pallas_call(
        functools.partial(_router_kernel, n_steps=grid[1], k_steps=grid[2]),
        grid=grid,
        in_specs=[
            pl.BlockSpec((bm, bk), lambda m, n, k: (m, k)),
            pl.BlockSpec((bk, bn), lambda m, n, k: (k, n)),
            pl.BlockSpec((1, bn), lambda m, n, k: (0, n)),
            pl.BlockSpec((bn, E), lambda m, n, k: (n, 0)),
            pl.BlockSpec((1, E), lambda m, n, k: (0, 0)),
        ],
        out_specs=pl.BlockSpec((bm, E), lambda m, n, k: (m, 0)),
        out_shape=jax.ShapeDtypeStruct((M, E), jnp.float32),
        scratch_shapes=[
            pltpu.VMEM((bm, bn), jnp.float32),
            pltpu.VMEM((bm, E), jnp.float32),
        ],
        compiler_params=pltpu.CompilerParams(
            dimension_semantics=("parallel", "arbitrary", "arbitrary")),
    )(x, W1, b1r, W2, b2r)
